# baseline (speedup 1.0000x reference)
import dataclasses
import functools

import jax
import jax.numpy as jnp
from jax import lax
from jax.experimental import pallas as pl
from jax.experimental.pallas import tpu as pltpu
from jax.experimental.pallas import tpu_sc as plsc

MEMROWS = 100000
BATCH = 16384
D = 784
NC, NS, NLANE = 2, 16, 16
NW = NC * NS
BPW = BATCH // NW
GW = 32
CH = 2048
WIN = 160
NWIN = MEMROWS // WIN

_sc_mesh = plsc.VectorSubcoreMesh(core_axis_name="c", subcore_axis_name="s")

_sc_params = pltpu.CompilerParams(use_tc_tiling_on_sc=False)
if "needs_layout_passes" in pltpu.CompilerParams.__dataclass_fields__:
    _sc_params = dataclasses.replace(_sc_params, needs_layout_passes=False)


def _wid():
    return lax.axis_index("c") * NS + lax.axis_index("s")


@functools.partial(
    pl.kernel,
    out_type=(
        jax.ShapeDtypeStruct((BATCH, D), jnp.float32),
        jax.ShapeDtypeStruct((MEMROWS,), jnp.int32),
    ),
    mesh=_sc_mesh,
    compiler_params=_sc_params,
    scratch_types=[
        pltpu.VMEM((MEMROWS,), jnp.int32),
        pltpu.VMEM((CH,), jnp.int32),
        pltpu.SemaphoreType.DMA,
    ],
)
def _k1(mem_hbm, idx_hbm, neg1_hbm, xrec_hbm, tab_hbm, tab_v, idx_v, sem_g):
    w = _wid()

    @pl.when(w == 0)
    def _():
        pltpu.sync_copy(neg1_hbm, tab_v)
        lane = lax.iota(jnp.int32, NLANE)

        for p in range(3):
            @pl.loop(0, BATCH, step=CH)
            def _(c):
                pltpu.sync_copy(idx_hbm.at[pl.ds(c, CH)], idx_v)

                @pl.loop(0, CH, step=NLANE)
                def _(v):
                    iv = idx_v[pl.ds(v, NLANE)]
                    bv = (c + v) + lane
                    if p == 0:
                        plsc.store_scatter(tab_v, [iv], bv)
                    else:
                        cur = plsc.load_gather(tab_v, [iv])
                        plsc.store_scatter(tab_v, [iv], bv, mask=cur < bv)

        pltpu.sync_copy(tab_v, tab_hbm)

    base = w * BPW
    pltpu.sync_copy(idx_hbm.at[pl.ds(base, BPW)], idx_v.at[pl.ds(0, BPW)])

    @pl.loop(0, BPW, step=GW)
    def _(g):
        for j0 in range(0, GW, NLANE):
            vals = idx_v[pl.ds(g + j0, NLANE)]
            for k in range(NLANE):
                pltpu.make_async_copy(
                    mem_hbm.at[pl.ds(vals[k], 1)],
                    xrec_hbm.at[pl.ds(base + g + j0 + k, 1)],
                    sem_g,
                ).start()

        for j in range(GW):
            pltpu.make_async_copy(
                mem_hbm.at[pl.ds(0, 1)],
                xrec_hbm.at[pl.ds(base + g + j, 1)],
                sem_g,
            ).wait()


@functools.partial(
    pl.kernel,
    out_type=jax.ShapeDtypeStruct((MEMROWS, D), jnp.float32),
    mesh=_sc_mesh,
    compiler_params=_sc_params,
    scratch_types=[
        pltpu.VMEM((WIN, D), jnp.float32),
        pltpu.VMEM((WIN,), jnp.int32),
        pltpu.SemaphoreType.DMA,
        pltpu.SemaphoreType.DMA,
    ],
)
def _k2(mem_hbm, x_hbm, tab_hbm, memnew_hbm, buf_v, tab_v, sem_w, sem_r):
    w = _wid()

    @pl.loop(w, NWIN, step=NW)
    def _(t):
        base = t * WIN
        cp_rows = pltpu.make_async_copy(mem_hbm.at[pl.ds(base, WIN)], buf_v, sem_w)
        cp_rows.start()
        pltpu.sync_copy(tab_hbm.at[pl.ds(base, WIN)], tab_v.at[pl.ds(0, WIN)])
        cp_rows.wait()

        for j0 in range(0, WIN, NLANE):
            tv = tab_v[pl.ds(j0, NLANE)]
            for k in range(NLANE):
                tk = tv[k]

                @pl.when(tk >= 0)
                def _():
                    pltpu.make_async_copy(
                        x_hbm.at[pl.ds(tk, 1)], buf_v.at[pl.ds(j0 + k, 1)], sem_r
                    ).start()

        for j0 in range(0, WIN, NLANE):
            tv = tab_v[pl.ds(j0, NLANE)]
            for k in range(NLANE):
                @pl.when(tv[k] >= 0)
                def _():
                    pltpu.make_async_copy(
                        x_hbm.at[pl.ds(0, 1)], buf_v.at[pl.ds(j0 + k, 1)], sem_r
                    ).wait()

        pltpu.sync_copy(buf_v, memnew_hbm.at[pl.ds(base, WIN)])


def _mlp_body(x_ref, xrec_ref, w1_ref, b1_ref, w2_ref, b2_ref, wrec_ref,
              brec_ref, wout_ref, bout_ref, out_ref):
    hp = lax.Precision.HIGHEST
    act = jnp.maximum(
        jnp.dot(x_ref[...], w1_ref[...], precision=hp,
                preferred_element_type=jnp.float32) + b1_ref[...], 0.0)
    a2 = (jnp.dot(act, w2_ref[...], precision=hp,
                  preferred_element_type=jnp.float32) + b2_ref[...]
          + jnp.dot(xrec_ref[...], wrec_ref[...], precision=hp,
                    preferred_element_type=jnp.float32) + brec_ref[...])
    a2 = jnp.maximum(a2, 0.0)
    out_ref[...] = jnp.dot(a2, wout_ref[...], precision=hp,
                           preferred_element_type=jnp.float32) + bout_ref[...]


def _mlp(x, xrec, w1, b1, w2, b2, wrec, brec, wout, bout):
    bm = 1024
    grid = (BATCH // bm,)
    full = lambda a: pl.BlockSpec(a.shape, lambda i: (0,) * a.ndim)
    return pl.pallas_call(
        _mlp_body,
        grid=grid,
        in_specs=[
            pl.BlockSpec((bm, D), lambda i: (i, 0)),
            pl.BlockSpec((bm, D), lambda i: (i, 0)),
            full(w1), full(b1), full(w2), full(b2),
            full(wrec), full(brec), full(wout), full(bout),
        ],
        out_specs=pl.BlockSpec((bm, 10), lambda i: (i, 0)),
        out_shape=jax.ShapeDtypeStruct((BATCH, 10), jnp.float32),
    )(x, xrec, w1, b1, w2, b2, wrec, brec, wout, bout)


def kernel(x_sensory, mem_vals, W1, b1, W2, b2, Wrec, brec, Wout, bout):
    h = lax.stop_gradient(jax.nn.relu(x_sensory @ W1 + b1))
    mult = jnp.arange(1, h.shape[1] + 1, dtype=jnp.float32) * 2654435.0
    code = jnp.floor(h * 8.0) @ mult
    idx = jnp.mod(jnp.abs(code), float(MEMROWS))
    idx = jnp.clip(idx.astype(jnp.int32), 0, MEMROWS - 1)

    neg1 = jnp.full((MEMROWS,), -1, jnp.int32)
    x_rec, tab = _k1(mem_vals, idx, neg1)
    mem_new = _k2(mem_vals, x_sensory, tab)

    out = _mlp(x_sensory, x_rec, W1, b1.reshape(1, -1), W2, b2.reshape(1, -1),
               Wrec, brec.reshape(1, -1), Wout, bout.reshape(1, -1))
    return out, mem_new
